# baseline (speedup 1.0000x reference)
import functools

import jax
import jax.numpy as jnp
from jax import lax
from jax.experimental import pallas as pl
from jax.experimental.pallas import tpu as pltpu
from jax.experimental.pallas import tpu_sc as plsc

_C = 81
_NC = 80
_ALPHA = 0.25
_BBOX_W, _OBJ_W, _CLS_W = 2.0, 1.0, 0.5

_B = 16
_T = 8
_K = _B * _T
_HW4, _W4 = 1024, 32
_HW5, _W5 = 256, 16
_NCORE, _NSUB = 2, 16
_NW = _NCORE * _NSUB
_TPW = _K // _NW
_GCH = 96


def _sc_gather_body(cls4_hbm, reg4_hbm, cls5_hbm, reg5_hbm, t4_hbm, t5_hbm,
                    g4_hbm, r4_hbm, g5_hbm, r5_hbm,
                    t4_v, t5_v, cell4_v, cell5_v, idx4_v, idxr4_v,
                    idx5_v, idxr5_v, out4_v, outr4_v, out5_v, outr5_v, sem):
    wid = lax.axis_index("s") * _NCORE + lax.axis_index("c")
    b = wid // 2
    half = wid % 2

    pltpu.sync_copy(t4_hbm.at[pl.ds(b * 40, 40)], t4_v.at[pl.ds(0, 40)])
    pltpu.sync_copy(t5_hbm.at[pl.ds(b * 40, 40)], t5_v.at[pl.ds(0, 40)])

    lane = lax.iota(jnp.int32, 16)
    tt = (lane & 3) + half * 4
    pos = tt * 5 + 1 + ((lane >> 2) & 1)

    txy4 = plsc.load_gather(t4_v, [pos])
    gxy4 = jnp.clip(txy4 * 32.0, 0.0, 31.0).astype(jnp.int32)
    txy5 = plsc.load_gather(t5_v, [pos])
    gxy5 = jnp.clip(txy5 * 16.0, 0.0, 15.0).astype(jnp.int32)

    t_of = jnp.minimum(lane, 3)
    cell4_v[pl.ds(16, 16)] = gxy4
    gx4 = plsc.load_gather(cell4_v, [t_of + 16])
    gy4 = plsc.load_gather(cell4_v, [t_of + 20])
    cell4_v[pl.ds(0, 16)] = gy4 * _W4 + gx4

    cell5_v[pl.ds(16, 16)] = gxy5
    gx5 = plsc.load_gather(cell5_v, [t_of + 16])
    gy5 = plsc.load_gather(cell5_v, [t_of + 20])
    cell5_v[pl.ds(0, 16)] = gy5 * _W5 + gx5

    for t in range(_TPW):
        tvec = jnp.full((16,), t, jnp.int32)
        bt4 = plsc.load_gather(cell4_v, [tvec])
        bt5 = plsc.load_gather(cell5_v, [tvec])
        for j in range(_GCH // 16):
            c = jnp.minimum(lane + j * 16, _C - 1)
            idx4_v[pl.ds((t * 6 + j) * 16, 16)] = b * (_C * _HW4) + bt4 + c * _HW4
            idx5_v[pl.ds((t * 6 + j) * 16, 16)] = b * (_C * _HW5) + bt5 + c * _HW5

    tg = lane >> 2
    rc = lane & 3
    btr4 = plsc.load_gather(cell4_v, [tg])
    btr5 = plsc.load_gather(cell5_v, [tg])
    idxr4_v[...] = b * (4 * _HW4) + btr4 + rc * _HW4
    idxr5_v[...] = b * (4 * _HW5) + btr5 + rc * _HW5

    descs = []
    for s in range(3):
        sl = pl.ds(s * 128, 128)
        descs.append(pltpu.async_copy(cls4_hbm.at[idx4_v.at[sl]],
                                      out4_v.at[sl], sem))
        descs.append(pltpu.async_copy(cls5_hbm.at[idx5_v.at[sl]],
                                      out5_v.at[sl], sem))
    descs.append(pltpu.async_copy(reg4_hbm.at[idxr4_v], outr4_v, sem))
    descs.append(pltpu.async_copy(reg5_hbm.at[idxr5_v], outr5_v, sem))
    for d in descs:
        d.wait()

    pltpu.sync_copy(out4_v, g4_hbm.at[pl.ds(wid * (_TPW * _GCH), _TPW * _GCH)])
    pltpu.sync_copy(out5_v, g5_hbm.at[pl.ds(wid * (_TPW * _GCH), _TPW * _GCH)])
    pltpu.sync_copy(outr4_v, r4_hbm.at[pl.ds(wid * 16, 16)])
    pltpu.sync_copy(outr5_v, r5_hbm.at[pl.ds(wid * 16, 16)])


def _sc_gather(cls4f, reg4f, cls5f, reg5f, t4f, t5f):
    return pl.kernel(
        _sc_gather_body,
        out_type=[
            jax.ShapeDtypeStruct((_K * _GCH,), jnp.float32),
            jax.ShapeDtypeStruct((_K * 4,), jnp.float32),
            jax.ShapeDtypeStruct((_K * _GCH,), jnp.float32),
            jax.ShapeDtypeStruct((_K * 4,), jnp.float32),
        ],
        mesh=plsc.VectorSubcoreMesh(core_axis_name="c", subcore_axis_name="s",
                                    num_cores=_NCORE, num_subcores=_NSUB),
        compiler_params=pltpu.CompilerParams(needs_layout_passes=False),
        scratch_types=[
            pltpu.VMEM((128,), jnp.float32),
            pltpu.VMEM((128,), jnp.float32),
            pltpu.VMEM((128,), jnp.int32),
            pltpu.VMEM((128,), jnp.int32),
            pltpu.VMEM((_TPW * _GCH,), jnp.int32),
            pltpu.VMEM((16,), jnp.int32),
            pltpu.VMEM((_TPW * _GCH,), jnp.int32),
            pltpu.VMEM((16,), jnp.int32),
            pltpu.VMEM((_TPW * _GCH,), jnp.float32),
            pltpu.VMEM((16,), jnp.float32),
            pltpu.VMEM((_TPW * _GCH,), jnp.float32),
            pltpu.VMEM((16,), jnp.float32),
            pltpu.SemaphoreType.DMA,
        ],
    )(cls4f, reg4f, cls5f, reg5f, t4f, t5f)


def _softplus(x):
    return jnp.maximum(x, 0.0) + jnp.log1p(jnp.exp(-jnp.abs(x)))


def _smooth_l1(pred, tgt):
    d = pred - tgt
    ad = jnp.abs(d)
    return jnp.where(ad < 1.0, 0.5 * d * d, ad - 0.5)


def _scale_terms(g, r, t2, t2t, w, hw, total_sp):
    tx = t2[:, 1:2] * w
    ty = t2[:, 2:3] * w
    tw = t2[:, 3:4] * w
    th = t2[:, 4:5] * w
    cls_ids = t2[:, 0:1].astype(jnp.int32)
    gx = jnp.clip(tx, 0.0, w - 1.0).astype(jnp.int32)
    gy = jnp.clip(ty, 0.0, w - 1.0).astype(jnp.int32)

    dx = 1.0 / (1.0 + jnp.exp(-r[:, 0:1]))
    dy = 1.0 / (1.0 + jnp.exp(-r[:, 1:2]))
    dw = jnp.exp(jnp.clip(r[:, 2:3], -4.0, 4.0))
    dh = jnp.exp(jnp.clip(r[:, 3:4], -4.0, 4.0))
    px = gx.astype(jnp.float32) + dx
    py = gy.astype(jnp.float32) + dy
    sl = (_smooth_l1(px - dw * 0.5, tx - tw * 0.5)
          + _smooth_l1(py - dh * 0.5, ty - th * 0.5)
          + _smooth_l1(px + dw * 0.5, tx + tw * 0.5)
          + _smooth_l1(py + dh * 0.5, ty + th * 0.5)) * 0.25
    lb = jnp.sum(sl)

    obj = g[:, 0:1]
    lo_pos = jnp.sum(_softplus(-obj))

    iota_i = lax.broadcasted_iota(jnp.int32, (_K, _K), 0)
    iota_j = lax.broadcasted_iota(jnp.int32, (_K, _K), 1)
    key = (iota_i >> 3) * hw + gy * w + gx
    txr = t2t[1:2, :] * w
    tyr = t2t[2:3, :] * w
    gxr = jnp.clip(txr, 0.0, w - 1.0).astype(jnp.int32)
    gyr = jnp.clip(tyr, 0.0, w - 1.0).astype(jnp.int32)
    keyr = (iota_j >> 3) * hw + gyr * w + gxr
    dup = jnp.max(((key == keyr) & (iota_j < iota_i))
                  .astype(jnp.float32), axis=1, keepdims=True)
    uniq = 1.0 - dup
    bg_sum = total_sp - jnp.sum(uniq * _softplus(obj))
    bg_cnt = float(_B * hw) - jnp.sum(uniq)
    lo = lo_pos + 0.05 * bg_sum / bg_cnt

    logits = g[:, 1:_C]
    iota_c = lax.broadcasted_iota(jnp.int32, (_K, _NC), 1)
    oh = (iota_c == cls_ids).astype(jnp.float32)
    bce = _softplus(logits) - logits * oh
    p = 1.0 / (1.0 + jnp.exp(-logits))
    pt = p * oh + (1.0 - p) * (1.0 - oh)
    one_m_pt = 1.0 - pt
    lc = jnp.sum(_ALPHA * one_m_pt * one_m_pt * bce) * (1.0 / _NC)
    return lb, lo, lc


def _tc_body(cls4_hbm, cls5_hbm, g4_ref, r4_ref, g5_ref, r5_ref,
             t4_ref, t5_ref, t4t_ref, t5t_ref, out_ref,
             ch0_4, ch0_5, sem4, sem5):
    c4 = pltpu.async_copy(cls4_hbm.at[:, pl.ds(0, _HW4)], ch0_4, sem4)
    c5 = pltpu.async_copy(cls5_hbm.at[:, pl.ds(0, _HW5)], ch0_5, sem5)
    c4.wait()
    c5.wait()
    sp4 = jnp.sum(_softplus(ch0_4[...]))
    sp5 = jnp.sum(_softplus(ch0_5[...]))
    lb4, lo4, lc4 = _scale_terms(g4_ref[...], r4_ref[...], t4_ref[...],
                                 t4t_ref[...], _W4, _HW4, sp4)
    lb5, lo5, lc5 = _scale_terms(g5_ref[...], r5_ref[...], t5_ref[...],
                                 t5t_ref[...], _W5, _HW5, sp5)
    n = float(2 * _K)
    total = (_BBOX_W * (lb4 + lb5) / n
             + _OBJ_W * (lo4 + lo5) / n
             + _CLS_W * (lc4 + lc5) / n)
    out_ref[0, 0] = total


def _tc_loss(cls4r, cls5r, g4, r4, g5, r5, t42, t52):
    t42t = jnp.swapaxes(t42, 0, 1)
    t52t = jnp.swapaxes(t52, 0, 1)

    out = pl.pallas_call(
        _tc_body,
        grid=(1,),
        in_specs=[
            pl.BlockSpec(memory_space=pl.ANY),
            pl.BlockSpec(memory_space=pl.ANY),
            pl.BlockSpec((_K, _GCH), lambda i: (0, 0)),
            pl.BlockSpec((_K, 4), lambda i: (0, 0)),
            pl.BlockSpec((_K, _GCH), lambda i: (0, 0)),
            pl.BlockSpec((_K, 4), lambda i: (0, 0)),
            pl.BlockSpec((_K, 5), lambda i: (0, 0)),
            pl.BlockSpec((_K, 5), lambda i: (0, 0)),
            pl.BlockSpec((5, _K), lambda i: (0, 0)),
            pl.BlockSpec((5, _K), lambda i: (0, 0)),
        ],
        out_specs=pl.BlockSpec(memory_space=pltpu.SMEM),
        out_shape=jax.ShapeDtypeStruct((1, 1), jnp.float32),
        scratch_shapes=[
            pltpu.VMEM((_B, _HW4), jnp.float32),
            pltpu.VMEM((_B, _HW5), jnp.float32),
            pltpu.SemaphoreType.DMA,
            pltpu.SemaphoreType.DMA,
        ],
    )(cls4r, cls5r, g4, r4, g5, r5, t42, t52, t42t, t52t)
    return out.reshape(())


def kernel(cls_p4, reg_p4, cls_p5, reg_p5, t4, t5):
    g4, r4, g5, r5 = _sc_gather(
        cls_p4.reshape(-1), reg_p4.reshape(-1),
        cls_p5.reshape(-1), reg_p5.reshape(-1),
        t4.reshape(-1), t5.reshape(-1))
    return _tc_loss(
        cls_p4.reshape(_B, _C * _HW4), cls_p5.reshape(_B, _C * _HW5),
        g4.reshape(_K, _GCH), r4.reshape(_K, 4),
        g5.reshape(_K, _GCH), r5.reshape(_K, 4),
        t4.reshape(_K, 5), t5.reshape(_K, 5))
